# baseline (speedup 1.0000x reference)
import functools

import jax
import jax.numpy as jnp
from jax import lax
from jax.experimental import pallas as pl
from jax.experimental.pallas import tpu as pltpu
from jax.experimental.pallas import tpu_sc as plsc

N = 10000
E = 320000
D = 128
R = 8
NB = 30

NC = 2
NS = 16
NW = NC * NS
C = 128
NCH = 80
EPW = NCH * C
EPAD = EPW * NW
CA = 512
NCHA = EPAD // NS // CA
TOTCH = EPAD // C
CN = 120
CS = TOTCH // NS - CN
RND = 40
NPAD = 10240
RPT = NPAD // NS

BN = 400
NBLK = N // BN

_mesh = plsc.VectorSubcoreMesh(core_axis_name="c", subcore_axis_name="s")


def _i16(v):
    return jnp.zeros((16,), jnp.int32) + v


HR = 640
HRPT = HR // NS


def _cw_body(ed_hbm, zero_hbm, w_hbm, g_hbm,
             acc_sh, hist, eda, edb, edc, edd, wbuf, gbuf, ivb, sA, sB):
    c = lax.axis_index("c")
    s = lax.axis_index("s")
    pltpu.sync_copy(zero_hbm.at[pl.ds(s * HRPT, HRPT)], acc_sh.at[pl.ds(s * HRPT, HRPT)])
    pltpu.sync_copy(zero_hbm.at[pl.ds(0, HR)], hist)

    base_a = s * (EPAD // NS)

    def issue_a(j, buf, sem):
        off = base_a + j * CA
        pltpu.async_copy(ed_hbm.at[:, pl.ds(off, CA)], buf, sem)

    def wait_a(buf, sem):
        pltpu.make_async_copy(ed_hbm.at[:, pl.ds(0, CA)], buf, sem).wait()

    def count(buf):
        for g in range(CA // 16):
            dv = buf[0, pl.ds(g * 16, 16)]
            ev = buf[1, pl.ds(g * 16, 16)]
            seg = dv * R + jnp.minimum(ev, R - 1)
            plsc.addupdate_scatter(hist,
                                   [lax.shift_right_logical(seg, 7),
                                    jnp.bitwise_and(seg, 127)],
                                   jnp.zeros((16,), jnp.float32) + 1.0,
                                   mask=ev < R)

    issue_a(0, eda, sA)

    def loop_a(jj, _):
        issue_a(jj + 1, edb, sB)
        wait_a(eda, sA)
        count(eda)
        issue_a(jnp.minimum(jj + 2, NCHA - 1), eda, sA)
        wait_a(edb, sB)
        count(edb)
        return 0

    lax.fori_loop(0, NCHA // 2, lambda t, x: loop_a(t * 2, x), 0)
    wait_a(eda, sA)
    plsc.subcore_barrier()

    def merge(p, _):
        def wr(g, _):
            ivb[0, pl.ds(g * 16, 16)] = lax.iota(jnp.int32, 16) + (p * C + g * 16)
            return 0

        lax.fori_loop(0, C // 16, wr, 0)
        pltpu.sync_copy(hist.at[pl.ds(p * C, C)], acc_sh.at[ivb.at[0]], add=True)
        return 0

    lax.fori_loop(0, HR // C, merge, 0)
    plsc.subcore_barrier()
    pltpu.sync_copy(acc_sh, hist)

    wid = c * NS + s
    base = wid * EPW

    def issue_b(j, buf, sem):
        off = base + j * C
        pltpu.async_copy(ed_hbm.at[:, pl.ds(off, C)], buf, sem)

    def wait_b(buf, sem):
        pltpu.make_async_copy(ed_hbm.at[:, pl.ds(0, C)], buf, sem).wait()

    def weights(buf, j):
        def grp(k, _):
            dv = buf[0, pl.ds(k * 16, 16)]
            ev = buf[1, pl.ds(k * 16, 16)]
            sv = buf[2, pl.ds(k * 16, 16)]
            seg = dv * R + jnp.minimum(ev, R - 1)
            cnt = plsc.load_gather(hist,
                                   [lax.shift_right_logical(seg, 7),
                                    jnp.bitwise_and(seg, 127)])
            wbuf[j, pl.ds(k * 16, 16)] = jnp.where(ev < R, 1.0 / jnp.maximum(cnt, 1.0), 0.0)
            gbuf[j, pl.ds(k * 16, 16)] = jnp.where(ev < R, ev, 0) * N + sv
            return 0

        lax.fori_loop(0, C // 16, grp, 0)

    issue_b(0, edc, sA)

    def loop_b(jj, _):
        issue_b(jj + 1, edd, sB)
        wait_b(edc, sA)
        weights(edc, jj)
        issue_b(jnp.minimum(jj + 2, NCH - 1), edc, sA)
        wait_b(edd, sB)
        weights(edd, jj + 1)
        return 0

    lax.fori_loop(0, NCH // 2, lambda t, x: loop_b(t * 2, x), 0)
    wait_b(edc, sA)
    pltpu.sync_copy(wbuf, w_hbm.at[pl.ds(wid * NCH, NCH)])
    pltpu.sync_copy(gbuf, g_hbm.at[pl.ds(wid * NCH, NCH)])


_sc_cw = pl.kernel(
    _cw_body,
    out_type=[jax.ShapeDtypeStruct((TOTCH, C), jnp.float32),
              jax.ShapeDtypeStruct((TOTCH, C), jnp.int32)],
    mesh=_mesh,
    scratch_types=[
        pltpu.VMEM_SHARED((HR, 128), jnp.float32),
        pltpu.VMEM((HR, 128), jnp.float32),
        pltpu.VMEM((3, CA), jnp.int32),
        pltpu.VMEM((3, CA), jnp.int32),
        pltpu.VMEM((3, C), jnp.int32),
        pltpu.VMEM((3, C), jnp.int32),
        pltpu.VMEM((NCH, C), jnp.float32),
        pltpu.VMEM((NCH, C), jnp.int32),
        pltpu.VMEM((1, C), jnp.int32),
        pltpu.SemaphoreType.DMA,
        pltpu.SemaphoreType.DMA,
    ],
    compiler_params=pltpu.CompilerParams(needs_layout_passes=False),
)


def _agg_body(tab_hbm, g_hbm, w_hbm, d_hbm, zero_hbm, out_hbm,
              acc_sh, gv, wv, dv, rowsA, rowsB, sA, sB):
    c = lax.axis_index("c")
    s = lax.axis_index("s")

    @pl.when(c == 0)
    def _():
        pltpu.sync_copy(zero_hbm.at[pl.ds(s * RPT, RPT)], acc_sh.at[pl.ds(s * RPT, RPT)])

    plsc.subcore_barrier()

    CT = TOTCH // NS

    def issue(j, rows, sem):
        pltpu.async_copy(tab_hbm.at[gv.at[j]], rows, sem)

    def wait(rows, sem):
        pltpu.make_async_copy(tab_hbm.at[gv.at[0]], rows, sem).wait()

    def scale_scatter(rows, j):
        def edge(i, _):
            wb = plsc.load_gather(wv, [_i16(j), _i16(i)])
            for t in range(D // 16):
                rows[i, pl.ds(t * 16, 16)] = rows[i, pl.ds(t * 16, 16)] * wb
            return 0

        lax.fori_loop(0, C, edge, 0)
        pltpu.sync_copy(rows, acc_sh.at[dv.at[j]], add=True)

    def rnd(r, _):
        cb = s * CT + r * RND
        pltpu.sync_copy(g_hbm.at[pl.ds(cb, RND)], gv)
        pltpu.sync_copy(w_hbm.at[pl.ds(cb, RND)], wv)
        pltpu.sync_copy(d_hbm.at[pl.ds(cb, RND)], dv)
        issue(0, rowsA, sA)

        def loop(jj, _):
            issue(jj + 1, rowsB, sB)
            wait(rowsA, sA)
            scale_scatter(rowsA, jj)
            issue(jnp.minimum(jj + 2, RND - 1), rowsA, sA)
            wait(rowsB, sB)
            scale_scatter(rowsB, jj + 1)
            return 0

        lax.fori_loop(0, RND // 2, lambda t, x: loop(t * 2, x), 0)
        wait(rowsA, sA)
        return 0

    nrounds = jnp.where(c == 0, CT // RND, 0)
    lax.fori_loop(0, nrounds, rnd, 0)
    plsc.subcore_barrier()

    @pl.when(c == 0)
    def _():
        pltpu.sync_copy(acc_sh.at[pl.ds(s * RPT, RPT)], out_hbm.at[pl.ds(s * RPT, RPT)])


_sc_agg = pl.kernel(
    _agg_body,
    out_type=jax.ShapeDtypeStruct((NPAD, D), jnp.float32),
    mesh=_mesh,
    scratch_types=[
        pltpu.VMEM_SHARED((NPAD, D), jnp.float32),
        pltpu.VMEM((RND, C), jnp.int32),
        pltpu.VMEM((RND, C), jnp.float32),
        pltpu.VMEM((RND, C), jnp.int32),
        pltpu.VMEM((C, D), jnp.float32),
        pltpu.VMEM((C, D), jnp.float32),
        pltpu.SemaphoreType.DMA,
        pltpu.SemaphoreType.DMA,
    ],
    compiler_params=pltpu.CompilerParams(needs_layout_passes=False),
)


def _wr_body(c0_ref, b0_ref, c1_ref, b1_ref, w0_ref, w1_ref):
    w0_ref[...] = jnp.dot(c0_ref[...], b0_ref[...], preferred_element_type=jnp.float32)
    w1_ref[...] = jnp.dot(c1_ref[...], b1_ref[...], preferred_element_type=jnp.float32)


_wr_kernel = pl.pallas_call(
    _wr_body,
    out_shape=[jax.ShapeDtypeStruct((R, D * D), jnp.float32),
               jax.ShapeDtypeStruct((R, D * D), jnp.float32)],
)


def _xw_body(x_ref, wr_ref, root_ref, bias_ref, xw_ref, hr_ref):
    xb = x_ref[...]
    for r in range(R):
        xw_ref[r] = jnp.dot(xb, wr_ref[r], preferred_element_type=jnp.float32)
    hr_ref[...] = jnp.dot(xb, root_ref[...], preferred_element_type=jnp.float32) + bias_ref[...]


_xw_kernel = pl.pallas_call(
    _xw_body,
    grid=(NBLK,),
    in_specs=[
        pl.BlockSpec((BN, D), lambda i: (i, 0)),
        pl.BlockSpec((R, D, D), lambda i: (0, 0, 0)),
        pl.BlockSpec((D, D), lambda i: (0, 0)),
        pl.BlockSpec((1, D), lambda i: (0, 0)),
    ],
    out_specs=[
        pl.BlockSpec((R, BN, D), lambda i: (0, i, 0)),
        pl.BlockSpec((BN, D), lambda i: (i, 0)),
    ],
    out_shape=[jax.ShapeDtypeStruct((R, N, D), jnp.float32),
               jax.ShapeDtypeStruct((N, D), jnp.float32)],
)


def _stats_body(aggp_ref, hr_ref, hpre_ref, mom_ref):
    i = pl.program_id(0)
    h = aggp_ref[...] + hr_ref[...]
    hpre_ref[...] = h
    s1 = jnp.sum(h, axis=0, keepdims=True)
    s2 = jnp.sum(h * h, axis=0, keepdims=True)
    upd = jnp.concatenate([s1, s2, jnp.zeros((6, D), jnp.float32)], axis=0)

    @pl.when(i == 0)
    def _():
        mom_ref[...] = upd

    @pl.when(i > 0)
    def _():
        mom_ref[...] += upd


_stats_kernel = pl.pallas_call(
    _stats_body,
    grid=(NBLK,),
    in_specs=[
        pl.BlockSpec((BN, D), lambda i: (i, 0)),
        pl.BlockSpec((BN, D), lambda i: (i, 0)),
    ],
    out_specs=[
        pl.BlockSpec((BN, D), lambda i: (i, 0)),
        pl.BlockSpec((8, D), lambda i: (0, 0)),
    ],
    out_shape=[jax.ShapeDtypeStruct((N, D), jnp.float32),
               jax.ShapeDtypeStruct((8, D), jnp.float32)],
)


def _bn_scale(mom_ref, gamma_ref, beta_ref):
    mu = mom_ref[0:1] / N
    var = mom_ref[1:2] / N - mu * mu
    a = gamma_ref[...] * lax.rsqrt(var + 1e-5)
    b = beta_ref[...] - mu * a
    return a, b


def _apply_body(hpre_ref, mom_ref, gamma_ref, beta_ref, wr_ref, root_ref, bias_ref,
                xw_ref, hr_ref):
    a, b = _bn_scale(mom_ref, gamma_ref, beta_ref)
    h1 = jax.nn.relu(hpre_ref[...] * a + b)
    for r in range(R):
        xw_ref[r] = jnp.dot(h1, wr_ref[r], preferred_element_type=jnp.float32)
    hr_ref[...] = jnp.dot(h1, root_ref[...], preferred_element_type=jnp.float32) + bias_ref[...]


_apply_kernel = pl.pallas_call(
    _apply_body,
    grid=(NBLK,),
    in_specs=[
        pl.BlockSpec((BN, D), lambda i: (i, 0)),
        pl.BlockSpec((8, D), lambda i: (0, 0)),
        pl.BlockSpec((1, D), lambda i: (0, 0)),
        pl.BlockSpec((1, D), lambda i: (0, 0)),
        pl.BlockSpec((R, D, D), lambda i: (0, 0, 0)),
        pl.BlockSpec((D, D), lambda i: (0, 0)),
        pl.BlockSpec((1, D), lambda i: (0, 0)),
    ],
    out_specs=[
        pl.BlockSpec((R, BN, D), lambda i: (0, i, 0)),
        pl.BlockSpec((BN, D), lambda i: (i, 0)),
    ],
    out_shape=[jax.ShapeDtypeStruct((R, N, D), jnp.float32),
               jax.ShapeDtypeStruct((N, D), jnp.float32)],
)


def _final_body(hpre_ref, mom_ref, gamma_ref, beta_ref, wf_ref, bf_ref, out_ref):
    a, b = _bn_scale(mom_ref, gamma_ref, beta_ref)
    h1 = jax.nn.relu(hpre_ref[...] * a + b)
    out_ref[...] = jnp.dot(h1, wf_ref[...], preferred_element_type=jnp.float32) + bf_ref[...]


_final_kernel = pl.pallas_call(
    _final_body,
    grid=(NBLK,),
    in_specs=[
        pl.BlockSpec((BN, D), lambda i: (i, 0)),
        pl.BlockSpec((8, D), lambda i: (0, 0)),
        pl.BlockSpec((1, D), lambda i: (0, 0)),
        pl.BlockSpec((1, D), lambda i: (0, 0)),
        pl.BlockSpec((D, D), lambda i: (0, 0)),
        pl.BlockSpec((1, D), lambda i: (0, 0)),
    ],
    out_specs=pl.BlockSpec((BN, D), lambda i: (i, 0)),
    out_shape=jax.ShapeDtypeStruct((N, D), jnp.float32),
)


def kernel(x, edge_index, edge_type, basis0, comp0, root0, bias0, gamma0, beta0,
           basis1, comp1, root1, bias1, gamma1, beta1, Wf, bf):
    pad = EPAD - E
    src_p = jnp.pad(edge_index[0], (0, pad))
    dst_p = jnp.pad(edge_index[1], (0, pad))
    et_p = jnp.pad(edge_type, (0, pad), constant_values=R)
    ed3 = jnp.stack([dst_p, et_p, src_p])
    dst3 = dst_p.reshape(TOTCH, C)
    zerosD = jnp.zeros((NPAD, D), jnp.float32)

    w_e, gidx = _sc_cw(ed3, zerosD)

    wr0f, wr1f = _wr_kernel(comp0, basis0.reshape(NB, D * D),
                            comp1, basis1.reshape(NB, D * D))
    wr0 = wr0f.reshape(R, D, D)
    wr1 = wr1f.reshape(R, D, D)

    xw0, hr0 = _xw_kernel(x, wr0, root0, bias0.reshape(1, D))
    agg0p = _sc_agg(xw0.reshape(R * N, D), gidx, w_e, dst3, zerosD)
    hpre0, mom0 = _stats_kernel(agg0p, hr0)

    xw1, hr1 = _apply_kernel(hpre0, mom0, gamma0.reshape(1, D), beta0.reshape(1, D),
                             wr1, root1, bias1.reshape(1, D))
    agg1p = _sc_agg(xw1.reshape(R * N, D), gidx, w_e, dst3, zerosD)
    hpre1, mom1 = _stats_kernel(agg1p, hr1)

    return _final_kernel(hpre1, mom1, gamma1.reshape(1, D), beta1.reshape(1, D),
                         Wf, bf.reshape(1, D))


# speedup vs baseline: 1.3358x; 1.3358x over previous
import functools

import jax
import jax.numpy as jnp
from jax import lax
from jax.experimental import pallas as pl
from jax.experimental.pallas import tpu as pltpu
from jax.experimental.pallas import tpu_sc as plsc

N = 10000
E = 320000
D = 128
R = 8
NB = 30

NC = 2
NS = 16
NW = NC * NS
C = 128
NCH = 80
EPW = NCH * C
EPAD = EPW * NW
CA = 512
NCHA = EPAD // NS // CA
TOTCH = EPAD // C
CN = 120
CS = TOTCH // NS - CN
RND = 40
NPAD = 10240
RPT = NPAD // NS

BN = 400
NBLK = N // BN

_mesh = plsc.VectorSubcoreMesh(core_axis_name="c", subcore_axis_name="s")


def _i16(v):
    return jnp.zeros((16,), jnp.int32) + v


HR = 640
HRPT = HR // NS


def _cw_body(ed_hbm, zero_hbm, w_hbm, g_hbm,
             acc_sh, hist, eda, edb, edc, edd, wbuf, gbuf, ivb, sA, sB):
    c = lax.axis_index("c")
    s = lax.axis_index("s")
    pltpu.sync_copy(zero_hbm.at[pl.ds(s * HRPT, HRPT)], acc_sh.at[pl.ds(s * HRPT, HRPT)])
    pltpu.sync_copy(zero_hbm.at[pl.ds(0, HR)], hist)

    base_a = s * (EPAD // NS)

    def issue_a(j, buf, sem):
        off = base_a + j * CA
        pltpu.async_copy(ed_hbm.at[:, pl.ds(off, CA)], buf, sem)

    def wait_a(buf, sem):
        pltpu.make_async_copy(ed_hbm.at[:, pl.ds(0, CA)], buf, sem).wait()

    def count(buf):
        for g in range(CA // 16):
            dv = buf[0, pl.ds(g * 16, 16)]
            ev = buf[1, pl.ds(g * 16, 16)]
            seg = dv * R + jnp.minimum(ev, R - 1)
            plsc.addupdate_scatter(hist,
                                   [lax.shift_right_logical(seg, 7),
                                    jnp.bitwise_and(seg, 127)],
                                   jnp.zeros((16,), jnp.float32) + 1.0,
                                   mask=ev < R)

    issue_a(0, eda, sA)

    def loop_a(jj, _):
        issue_a(jj + 1, edb, sB)
        wait_a(eda, sA)
        count(eda)
        issue_a(jnp.minimum(jj + 2, NCHA - 1), eda, sA)
        wait_a(edb, sB)
        count(edb)
        return 0

    lax.fori_loop(0, NCHA // 2, lambda t, x: loop_a(t * 2, x), 0)
    wait_a(eda, sA)
    plsc.subcore_barrier()

    def merge(p, _):
        def wr(g, _):
            ivb[0, pl.ds(g * 16, 16)] = lax.iota(jnp.int32, 16) + (p * C + g * 16)
            return 0

        lax.fori_loop(0, C // 16, wr, 0)
        pltpu.sync_copy(hist.at[pl.ds(p * C, C)], acc_sh.at[ivb.at[0]], add=True)
        return 0

    lax.fori_loop(0, HR // C, merge, 0)
    plsc.subcore_barrier()
    pltpu.sync_copy(acc_sh, hist)

    wid = c * NS + s
    base = wid * EPW

    def issue_b(j, buf, sem):
        off = base + j * C
        pltpu.async_copy(ed_hbm.at[:, pl.ds(off, C)], buf, sem)

    def wait_b(buf, sem):
        pltpu.make_async_copy(ed_hbm.at[:, pl.ds(0, C)], buf, sem).wait()

    def weights(buf, j):
        def grp(k, _):
            dv = buf[0, pl.ds(k * 16, 16)]
            ev = buf[1, pl.ds(k * 16, 16)]
            sv = buf[2, pl.ds(k * 16, 16)]
            seg = dv * R + jnp.minimum(ev, R - 1)
            cnt = plsc.load_gather(hist,
                                   [lax.shift_right_logical(seg, 7),
                                    jnp.bitwise_and(seg, 127)])
            wbuf[j, pl.ds(k * 16, 16)] = jnp.where(ev < R, 1.0 / jnp.maximum(cnt, 1.0), 0.0)
            gbuf[j, pl.ds(k * 16, 16)] = jnp.where(ev < R, ev, 0) * N + sv
            return 0

        lax.fori_loop(0, C // 16, grp, 0)

    issue_b(0, edc, sA)

    def loop_b(jj, _):
        issue_b(jj + 1, edd, sB)
        wait_b(edc, sA)
        weights(edc, jj)
        issue_b(jnp.minimum(jj + 2, NCH - 1), edc, sA)
        wait_b(edd, sB)
        weights(edd, jj + 1)
        return 0

    lax.fori_loop(0, NCH // 2, lambda t, x: loop_b(t * 2, x), 0)
    wait_b(edc, sA)
    pltpu.sync_copy(wbuf, w_hbm.at[pl.ds(wid * NCH, NCH)])
    pltpu.sync_copy(gbuf, g_hbm.at[pl.ds(wid * NCH, NCH)])


_sc_cw = pl.kernel(
    _cw_body,
    out_type=[jax.ShapeDtypeStruct((TOTCH, C), jnp.float32),
              jax.ShapeDtypeStruct((TOTCH, C), jnp.int32)],
    mesh=_mesh,
    scratch_types=[
        pltpu.VMEM_SHARED((HR, 128), jnp.float32),
        pltpu.VMEM((HR, 128), jnp.float32),
        pltpu.VMEM((3, CA), jnp.int32),
        pltpu.VMEM((3, CA), jnp.int32),
        pltpu.VMEM((3, C), jnp.int32),
        pltpu.VMEM((3, C), jnp.int32),
        pltpu.VMEM((NCH, C), jnp.float32),
        pltpu.VMEM((NCH, C), jnp.int32),
        pltpu.VMEM((1, C), jnp.int32),
        pltpu.SemaphoreType.DMA,
        pltpu.SemaphoreType.DMA,
    ],
    compiler_params=pltpu.CompilerParams(needs_layout_passes=False),
)


def _agg_body(tab_hbm, g_hbm, w_hbm, d_hbm, zero_hbm, out_hbm,
              acc_sh, gv, wv, dv, rowsA, rowsB, sA, sB):
    c = lax.axis_index("c")
    s = lax.axis_index("s")
    pltpu.sync_copy(zero_hbm.at[pl.ds(s * RPT, RPT)], acc_sh.at[pl.ds(s * RPT, RPT)])
    plsc.subcore_barrier()

    nrounds = jnp.where(c == 0, CN // RND, CS // RND)
    tile_base = jnp.where(c == 0, s * CN, NS * CN + s * CS)

    def issue(j, rows, sem):
        pltpu.async_copy(tab_hbm.at[gv.at[j]], rows, sem)

    def wait(rows, sem):
        pltpu.make_async_copy(tab_hbm.at[gv.at[0]], rows, sem).wait()

    def scale_scatter(rows, j):
        def edge(i, _):
            wb = plsc.load_gather(wv, [_i16(j), _i16(i)])
            for t in range(D // 16):
                rows[i, pl.ds(t * 16, 16)] = rows[i, pl.ds(t * 16, 16)] * wb
            return 0

        lax.fori_loop(0, C, edge, 0)
        pltpu.sync_copy(rows, acc_sh.at[dv.at[j]], add=True)

    def rnd(r, _):
        cb = tile_base + r * RND
        pltpu.sync_copy(g_hbm.at[pl.ds(cb, RND)], gv)
        pltpu.sync_copy(w_hbm.at[pl.ds(cb, RND)], wv)
        pltpu.sync_copy(d_hbm.at[pl.ds(cb, RND)], dv)
        issue(0, rowsA, sA)

        def loop(jj, _):
            issue(jj + 1, rowsB, sB)
            wait(rowsA, sA)
            scale_scatter(rowsA, jj)
            issue(jnp.minimum(jj + 2, RND - 1), rowsA, sA)
            wait(rowsB, sB)
            scale_scatter(rowsB, jj + 1)
            return 0

        lax.fori_loop(0, RND // 2, lambda t, x: loop(t * 2, x), 0)
        wait(rowsA, sA)
        return 0

    lax.fori_loop(0, nrounds, rnd, 0)
    plsc.subcore_barrier()
    pltpu.sync_copy(acc_sh.at[pl.ds(s * RPT, RPT)], out_hbm.at[c].at[pl.ds(s * RPT, RPT)])


_sc_agg = pl.kernel(
    _agg_body,
    out_type=jax.ShapeDtypeStruct((NC, NPAD, D), jnp.float32),
    mesh=_mesh,
    scratch_types=[
        pltpu.VMEM_SHARED((NPAD, D), jnp.float32),
        pltpu.VMEM((RND, C), jnp.int32),
        pltpu.VMEM((RND, C), jnp.float32),
        pltpu.VMEM((RND, C), jnp.int32),
        pltpu.VMEM((C, D), jnp.float32),
        pltpu.VMEM((C, D), jnp.float32),
        pltpu.SemaphoreType.DMA,
        pltpu.SemaphoreType.DMA,
    ],
    compiler_params=pltpu.CompilerParams(needs_layout_passes=False),
)


def _wr_body(c0_ref, b0_ref, c1_ref, b1_ref, w0_ref, w1_ref):
    w0_ref[...] = jnp.dot(c0_ref[...], b0_ref[...], preferred_element_type=jnp.float32)
    w1_ref[...] = jnp.dot(c1_ref[...], b1_ref[...], preferred_element_type=jnp.float32)


_wr_kernel = pl.pallas_call(
    _wr_body,
    out_shape=[jax.ShapeDtypeStruct((R, D * D), jnp.float32),
               jax.ShapeDtypeStruct((R, D * D), jnp.float32)],
)


def _xw_body(x_ref, wr_ref, root_ref, bias_ref, xw_ref, hr_ref):
    xb = x_ref[...]
    for r in range(R):
        xw_ref[r] = jnp.dot(xb, wr_ref[r], preferred_element_type=jnp.float32)
    hr_ref[...] = jnp.dot(xb, root_ref[...], preferred_element_type=jnp.float32) + bias_ref[...]


_xw_kernel = pl.pallas_call(
    _xw_body,
    grid=(NBLK,),
    in_specs=[
        pl.BlockSpec((BN, D), lambda i: (i, 0)),
        pl.BlockSpec((R, D, D), lambda i: (0, 0, 0)),
        pl.BlockSpec((D, D), lambda i: (0, 0)),
        pl.BlockSpec((1, D), lambda i: (0, 0)),
    ],
    out_specs=[
        pl.BlockSpec((R, BN, D), lambda i: (0, i, 0)),
        pl.BlockSpec((BN, D), lambda i: (i, 0)),
    ],
    out_shape=[jax.ShapeDtypeStruct((R, N, D), jnp.float32),
               jax.ShapeDtypeStruct((N, D), jnp.float32)],
)


def _stats_body(aggp_ref, hr_ref, hpre_ref, mom_ref):
    i = pl.program_id(0)
    h = aggp_ref[0] + aggp_ref[1] + hr_ref[...]
    hpre_ref[...] = h
    s1 = jnp.sum(h, axis=0, keepdims=True)
    s2 = jnp.sum(h * h, axis=0, keepdims=True)
    upd = jnp.concatenate([s1, s2, jnp.zeros((6, D), jnp.float32)], axis=0)

    @pl.when(i == 0)
    def _():
        mom_ref[...] = upd

    @pl.when(i > 0)
    def _():
        mom_ref[...] += upd


_stats_kernel = pl.pallas_call(
    _stats_body,
    grid=(NBLK,),
    in_specs=[
        pl.BlockSpec((NC, BN, D), lambda i: (0, i, 0)),
        pl.BlockSpec((BN, D), lambda i: (i, 0)),
    ],
    out_specs=[
        pl.BlockSpec((BN, D), lambda i: (i, 0)),
        pl.BlockSpec((8, D), lambda i: (0, 0)),
    ],
    out_shape=[jax.ShapeDtypeStruct((N, D), jnp.float32),
               jax.ShapeDtypeStruct((8, D), jnp.float32)],
)


def _bn_scale(mom_ref, gamma_ref, beta_ref):
    mu = mom_ref[0:1] / N
    var = mom_ref[1:2] / N - mu * mu
    a = gamma_ref[...] * lax.rsqrt(var + 1e-5)
    b = beta_ref[...] - mu * a
    return a, b


def _apply_body(hpre_ref, mom_ref, gamma_ref, beta_ref, wr_ref, root_ref, bias_ref,
                xw_ref, hr_ref):
    a, b = _bn_scale(mom_ref, gamma_ref, beta_ref)
    h1 = jax.nn.relu(hpre_ref[...] * a + b)
    for r in range(R):
        xw_ref[r] = jnp.dot(h1, wr_ref[r], preferred_element_type=jnp.float32)
    hr_ref[...] = jnp.dot(h1, root_ref[...], preferred_element_type=jnp.float32) + bias_ref[...]


_apply_kernel = pl.pallas_call(
    _apply_body,
    grid=(NBLK,),
    in_specs=[
        pl.BlockSpec((BN, D), lambda i: (i, 0)),
        pl.BlockSpec((8, D), lambda i: (0, 0)),
        pl.BlockSpec((1, D), lambda i: (0, 0)),
        pl.BlockSpec((1, D), lambda i: (0, 0)),
        pl.BlockSpec((R, D, D), lambda i: (0, 0, 0)),
        pl.BlockSpec((D, D), lambda i: (0, 0)),
        pl.BlockSpec((1, D), lambda i: (0, 0)),
    ],
    out_specs=[
        pl.BlockSpec((R, BN, D), lambda i: (0, i, 0)),
        pl.BlockSpec((BN, D), lambda i: (i, 0)),
    ],
    out_shape=[jax.ShapeDtypeStruct((R, N, D), jnp.float32),
               jax.ShapeDtypeStruct((N, D), jnp.float32)],
)


def _final_body(hpre_ref, mom_ref, gamma_ref, beta_ref, wf_ref, bf_ref, out_ref):
    a, b = _bn_scale(mom_ref, gamma_ref, beta_ref)
    h1 = jax.nn.relu(hpre_ref[...] * a + b)
    out_ref[...] = jnp.dot(h1, wf_ref[...], preferred_element_type=jnp.float32) + bf_ref[...]


_final_kernel = pl.pallas_call(
    _final_body,
    grid=(NBLK,),
    in_specs=[
        pl.BlockSpec((BN, D), lambda i: (i, 0)),
        pl.BlockSpec((8, D), lambda i: (0, 0)),
        pl.BlockSpec((1, D), lambda i: (0, 0)),
        pl.BlockSpec((1, D), lambda i: (0, 0)),
        pl.BlockSpec((D, D), lambda i: (0, 0)),
        pl.BlockSpec((1, D), lambda i: (0, 0)),
    ],
    out_specs=pl.BlockSpec((BN, D), lambda i: (i, 0)),
    out_shape=jax.ShapeDtypeStruct((N, D), jnp.float32),
)


def kernel(x, edge_index, edge_type, basis0, comp0, root0, bias0, gamma0, beta0,
           basis1, comp1, root1, bias1, gamma1, beta1, Wf, bf):
    pad = EPAD - E
    src_p = jnp.pad(edge_index[0], (0, pad))
    dst_p = jnp.pad(edge_index[1], (0, pad))
    et_p = jnp.pad(edge_type, (0, pad), constant_values=R)
    ed3 = jnp.stack([dst_p, et_p, src_p])
    dst3 = dst_p.reshape(TOTCH, C)
    zerosD = jnp.zeros((NPAD, D), jnp.float32)

    w_e, gidx = _sc_cw(ed3, zerosD)

    wr0f, wr1f = _wr_kernel(comp0, basis0.reshape(NB, D * D),
                            comp1, basis1.reshape(NB, D * D))
    wr0 = wr0f.reshape(R, D, D)
    wr1 = wr1f.reshape(R, D, D)

    xw0, hr0 = _xw_kernel(x, wr0, root0, bias0.reshape(1, D))
    agg0p = _sc_agg(xw0.reshape(R * N, D), gidx, w_e, dst3, zerosD)
    hpre0, mom0 = _stats_kernel(agg0p, hr0)

    xw1, hr1 = _apply_kernel(hpre0, mom0, gamma0.reshape(1, D), beta0.reshape(1, D),
                             wr1, root1, bias1.reshape(1, D))
    agg1p = _sc_agg(xw1.reshape(R * N, D), gidx, w_e, dst3, zerosD)
    hpre1, mom1 = _stats_kernel(agg1p, hr1)

    return _final_kernel(hpre1, mom1, gamma1.reshape(1, D), beta1.reshape(1, D),
                         Wf, bf.reshape(1, D))
